# baseline (speedup 1.0000x reference)
import jax
import jax.numpy as jnp
from jax.experimental import pallas as pl
from jax.experimental.pallas import tpu as pltpu


_BM = 512
_LAG = 2
_W1_CHUNKS = 8


def _mlp_block(x_ref, w1_hbm, b1_ref, w2_hbm, b2_ref, o_ref,
               h_scr, w1_scr, w2_scr, w1_sems, w2_sem):
    i = pl.program_id(0)
    nsteps = pl.num_programs(0)
    n1 = w1_scr.shape[0]
    ch = n1 // _W1_CHUNKS

    def w1_copy(nb):
        sl = pl.ds(nb * ch, ch)
        return pltpu.make_async_copy(w1_hbm.at[sl, :], w1_scr.at[sl, :],
                                     w1_sems.at[nb])

    w2_copy = pltpu.make_async_copy(w2_hbm, w2_scr, w2_sem)

    @pl.when(i == 0)
    def _start_weight_dmas():
        for nb in range(_W1_CHUNKS):
            w1_copy(nb).start()
        w2_copy.start()

    @pl.when(i == 0)
    def _layer1_first():
        xb = x_ref[...].astype(jnp.bfloat16)
        for nb in range(_W1_CHUNKS):
            w1_copy(nb).wait()
            sl = pl.ds(nb * ch, ch)
            h = jax.lax.dot_general(
                xb, w1_scr[sl, :], (((1,), (1,)), ((), ())),
                preferred_element_type=jnp.float32)
            h = jnp.maximum(h + b1_ref[:, sl], 0.0)
            h_scr[0, :, sl] = h.astype(jnp.bfloat16)

    @pl.when(jnp.logical_and(i > 0, i < nsteps - _LAG))
    def _layer1():
        xb = x_ref[...].astype(jnp.bfloat16)
        h = jax.lax.dot_general(
            xb, w1_scr[...], (((1,), (1,)), ((), ())),
            preferred_element_type=jnp.float32)
        h = jnp.maximum(h + b1_ref[...], 0.0)
        h_scr[i % (_LAG + 1)] = h.astype(jnp.bfloat16)

    @pl.when(i == _LAG)
    def _wait_w2():
        w2_copy.wait()

    @pl.when(i >= _LAG)
    def _layer2():
        hb = h_scr[(i - _LAG) % (_LAG + 1)]
        o = jax.lax.dot_general(
            hb, w2_scr[...], (((1,), (1,)), ((), ())),
            preferred_element_type=jnp.float32)
        o_ref[...] = o + b2_ref[...]


def kernel(input, W1, b1, W2, b2):
    M, K = input.shape
    N1, _ = W1.shape
    N2, _ = W2.shape
    nblocks = M // _BM
    grid = (nblocks + _LAG,)
    last = nblocks - 1
    return pl.pallas_call(
        _mlp_block,
        grid=grid,
        in_specs=[
            pl.BlockSpec((_BM, K), lambda i: (jnp.minimum(i, last), 0)),
            pl.BlockSpec(memory_space=pl.ANY),
            pl.BlockSpec((1, N1), lambda i: (0, 0)),
            pl.BlockSpec(memory_space=pl.ANY),
            pl.BlockSpec((1, N2), lambda i: (0, 0)),
        ],
        out_specs=pl.BlockSpec((_BM, N2), lambda i: (jnp.maximum(i - _LAG, 0), 0)),
        out_shape=jax.ShapeDtypeStruct((M, N2), jnp.float32),
        scratch_shapes=[
            pltpu.VMEM((_LAG + 1, _BM, N1), jnp.bfloat16),
            pltpu.VMEM((N1, K), jnp.float32),
            pltpu.VMEM((N2, N1), jnp.float32),
            pltpu.SemaphoreType.DMA((_W1_CHUNKS,)),
            pltpu.SemaphoreType.DMA,
        ],
    )(input, W1, b1.reshape(1, N1), W2, b2.reshape(1, N2))


# speedup vs baseline: 1.0320x; 1.0320x over previous
import jax
import jax.numpy as jnp
from jax.experimental import pallas as pl
from jax.experimental.pallas import tpu as pltpu


_BM = 512
_LAG = 2


def _mlp_block(x_ref, w1_ref, b1_ref, w2_hbm_ref, b2_ref, o_ref,
               h_scr, w2_vmem, dma_sem):
    i = pl.program_id(0)
    nsteps = pl.num_programs(0)
    w2_copy = pltpu.make_async_copy(w2_hbm_ref, w2_vmem, dma_sem)

    @pl.when(i == 0)
    def _start_w2():
        w2_copy.start()

    @pl.when(i < nsteps - _LAG)
    def _layer1():
        xb = x_ref[...].astype(jnp.bfloat16)
        h = jax.lax.dot_general(
            xb, w1_ref[...], (((1,), (1,)), ((), ())),
            preferred_element_type=jnp.float32)
        h = jnp.maximum(h + b1_ref[...], 0.0)
        h_scr[i % (_LAG + 1)] = h.astype(jnp.bfloat16)

    @pl.when(i == _LAG)
    def _wait_w2():
        w2_copy.wait()

    @pl.when(i >= _LAG)
    def _layer2():
        hb = h_scr[(i - _LAG) % (_LAG + 1)]
        o = jax.lax.dot_general(
            hb, w2_vmem[...], (((1,), (1,)), ((), ())),
            preferred_element_type=jnp.float32)
        o_ref[...] = o + b2_ref[...]


def kernel(input, W1, b1, W2, b2):
    M, K = input.shape
    N1, _ = W1.shape
    N2, _ = W2.shape
    nblocks = M // _BM
    grid = (nblocks + _LAG,)
    last = nblocks - 1
    return pl.pallas_call(
        _mlp_block,
        grid=grid,
        in_specs=[
            pl.BlockSpec((_BM, K), lambda i: (jnp.minimum(i, last), 0)),
            pl.BlockSpec((N1, K), lambda i: (0, 0)),
            pl.BlockSpec((1, N1), lambda i: (0, 0)),
            pl.BlockSpec(memory_space=pl.ANY),
            pl.BlockSpec((1, N2), lambda i: (0, 0)),
        ],
        out_specs=pl.BlockSpec((_BM, N2), lambda i: (jnp.maximum(i - _LAG, 0), 0)),
        out_shape=jax.ShapeDtypeStruct((M, N2), jnp.float32),
        scratch_shapes=[
            pltpu.VMEM((_LAG + 1, _BM, N1), jnp.bfloat16),
            pltpu.VMEM((N2, N1), jnp.float32),
            pltpu.SemaphoreType.DMA,
        ],
        compiler_params=pltpu.CompilerParams(
            vmem_limit_bytes=63 * 1024 * 1024),
    )(input, W1, b1.reshape(1, N1), W2, b2.reshape(1, N2))
